# baseline (speedup 1.0000x reference)
import functools

import jax
import jax.numpy as jnp
from jax import lax
from jax.experimental import pallas as pl
from jax.experimental.pallas import tpu as pltpu
from jax.experimental.pallas import tpu_sc as plsc

V = 100
D = 128
B = 16384
VARIATIONAL_WEIGHT = 0.001

NC = 2
NS = 16
L = 16
NW = NC * NS
B_PER_W = B // NW
CHUNK = 128
N_CHUNKS = B_PER_W // CHUNK


def _sc_body(idx_hbm, eps_hbm, mean_hbm, lv_hbm, code_hbm, kld_hbm,
             mean_t, lv_t, idx_v, eps_v, code_v, kld_v, sem_e):
    wid = lax.axis_index("s") * NC + lax.axis_index("c")
    pltpu.sync_copy(mean_hbm, mean_t)
    pltpu.sync_copy(lv_hbm, lv_t)

    lane = lax.iota(jnp.int32, L)

    def chunk_body(c, carry):
        base = wid * B_PER_W + c * CHUNK
        pltpu.sync_copy(idx_hbm.at[pl.ds(base, CHUNK)], idx_v)
        pltpu.async_copy(eps_hbm.at[pl.ds(base * D, CHUNK * D)], eps_v,
                         sem_e).wait()

        def group_body(g, carry):
            gbase = g * L
            tbl_idx = idx_v[pl.ds(gbase, L)] * D
            row_idx = (gbase + lane) * D
            acc = jnp.zeros((L,), jnp.float32)
            for col in range(D):
                tc = tbl_idx + col
                rc = row_idx + col
                m = plsc.load_gather(mean_t, [tc])
                lv = plsc.load_gather(lv_t, [tc])
                e = plsc.load_gather(eps_v, [rc])
                s = jnp.exp(0.5 * lv)
                plsc.store_scatter(code_v, [rc], e * s + m)
                acc = acc + (1.0 + lv - m * m - s * s)
            kld_v[pl.ds(gbase, L)] = acc * (-0.5 * VARIATIONAL_WEIGHT)
            return carry

        lax.fori_loop(0, CHUNK // L, group_body, 0)
        pltpu.sync_copy(code_v, code_hbm.at[pl.ds(base * D, CHUNK * D)])
        pltpu.sync_copy(kld_v, kld_hbm.at[pl.ds(base, CHUNK)])
        return carry

    lax.fori_loop(0, N_CHUNKS, chunk_body, 0)


@jax.jit
def _run(indices, eps, mean_table, logvar_table):
    mesh = plsc.VectorSubcoreMesh(core_axis_name="c", subcore_axis_name="s")
    f = functools.partial(
        pl.kernel,
        out_type=(
            jax.ShapeDtypeStruct((B * D,), jnp.float32),
            jax.ShapeDtypeStruct((B,), jnp.float32),
        ),
        mesh=mesh,
        compiler_params=pltpu.CompilerParams(needs_layout_passes=False),
        scratch_types=[
            pltpu.VMEM((V * D,), jnp.float32),
            pltpu.VMEM((V * D,), jnp.float32),
            pltpu.VMEM((CHUNK,), jnp.int32),
            pltpu.VMEM((CHUNK * D,), jnp.float32),
            pltpu.VMEM((CHUNK * D,), jnp.float32),
            pltpu.VMEM((CHUNK,), jnp.float32),
            pltpu.SemaphoreType.DMA,
        ],
    )(_sc_body)
    code_flat, kld = f(indices, eps.reshape(B * D), mean_table.reshape(V * D),
                       logvar_table.reshape(V * D))
    return code_flat.reshape(B, D), kld


def kernel(indices, eps, mean_table, logvar_table):
    code, kld = _run(indices.astype(jnp.int32), eps, mean_table, logvar_table)
    return (code, kld)


# speedup vs baseline: 3.8823x; 3.8823x over previous
import functools

import jax
import jax.numpy as jnp
from jax import lax
from jax.experimental import pallas as pl
from jax.experimental.pallas import tpu as pltpu
from jax.experimental.pallas import tpu_sc as plsc

V = 100
VP = 128
D = 128
B = 16384
VARIATIONAL_WEIGHT = 0.001

NC = 2
NS = 16
L = 16
NW = NC * NS
B_PER_W = B // NW
CHUNK = 128
N_CHUNKS = B_PER_W // CHUNK
R_PER_S = VP // NS


def _sc_body(idx_hbm, eps_hbm, mean_hbm, lv_hbm, code_hbm, kld_hbm,
             std_sh, g_sh,
             mean_t, std_t, g_v, idx_v, eps_v, code_v, kld_v,
             a_st, b_st, g_st, sem_e):
    sid = lax.axis_index("s")
    wid = sid * NC + lax.axis_index("c")
    lane = lax.iota(jnp.int32, L)

    trow = sid * R_PER_S
    pltpu.sync_copy(lv_hbm.at[pl.ds(trow * D, R_PER_S * D)], a_st)
    pltpu.sync_copy(mean_hbm.at[pl.ds(trow * D, R_PER_S * D)], b_st)

    def prow(r2, g_acc):
        acc = jnp.zeros((L,), jnp.float32)
        for j in range(D // L):
            sl = pl.ds(r2 * D + j * L, L)
            lv = a_st[sl]
            m = b_st[sl]
            s = jnp.exp(0.5 * lv)
            a_st[sl] = s
            acc = acc + (1.0 + lv - m * m - s * s)
        for _ in range(4):
            acc = acc + lax.rev(acc, (0,))
        return jnp.where(lane == r2, acc * (-0.5 * VARIATIONAL_WEIGHT), g_acc)

    g_st[...] = lax.fori_loop(0, R_PER_S, prow, jnp.zeros((L,), jnp.float32))
    pltpu.sync_copy(a_st, std_sh.at[pl.ds(trow * D, R_PER_S * D)])
    pltpu.sync_copy(g_st.at[pl.ds(0, R_PER_S)],
                    g_sh.at[pl.ds(trow, R_PER_S)])
    plsc.subcore_barrier()
    pltpu.sync_copy(std_sh, std_t)
    pltpu.sync_copy(g_sh, g_v)
    pltpu.sync_copy(mean_hbm, mean_t)

    def chunk_body(c, carry):
        base = wid * B_PER_W + c * CHUNK
        pltpu.sync_copy(idx_hbm.at[pl.ds(base, CHUNK)], idx_v)
        pltpu.async_copy(eps_hbm.at[pl.ds(base * D, CHUNK * D)], eps_v,
                         sem_e).wait()

        def group_body(g, carry):
            gbase = g * L
            tbl_vec = idx_v[pl.ds(gbase, L)]
            kld_v[pl.ds(gbase, L)] = plsc.load_gather(g_v, [tbl_vec])
            for rr in range(L):
                toff = tbl_vec[rr] * D
                roff = (gbase + rr) * D
                for j in range(D // L):
                    tsl = pl.ds(toff + j * L, L)
                    rsl = pl.ds(roff + j * L, L)
                    code_v[rsl] = eps_v[rsl] * std_t[tsl] + mean_t[tsl]
            return carry

        lax.fori_loop(0, CHUNK // L, group_body, 0)
        pltpu.sync_copy(code_v, code_hbm.at[pl.ds(base * D, CHUNK * D)])
        pltpu.sync_copy(kld_v, kld_hbm.at[pl.ds(base, CHUNK)])
        return carry

    lax.fori_loop(0, N_CHUNKS, chunk_body, 0)


@jax.jit
def _run(indices, eps, mean_table, logvar_table):
    mesh = plsc.VectorSubcoreMesh(core_axis_name="c", subcore_axis_name="s")
    f = functools.partial(
        pl.kernel,
        out_type=(
            jax.ShapeDtypeStruct((B * D,), jnp.float32),
            jax.ShapeDtypeStruct((B,), jnp.float32),
        ),
        mesh=mesh,
        compiler_params=pltpu.CompilerParams(needs_layout_passes=False),
        scratch_types=[
            pltpu.VMEM_SHARED((VP * D,), jnp.float32),
            pltpu.VMEM_SHARED((VP,), jnp.float32),
            pltpu.VMEM((VP * D,), jnp.float32),
            pltpu.VMEM((VP * D,), jnp.float32),
            pltpu.VMEM((VP,), jnp.float32),
            pltpu.VMEM((CHUNK,), jnp.int32),
            pltpu.VMEM((CHUNK * D,), jnp.float32),
            pltpu.VMEM((CHUNK * D,), jnp.float32),
            pltpu.VMEM((CHUNK,), jnp.float32),
            pltpu.VMEM((R_PER_S * D,), jnp.float32),
            pltpu.VMEM((R_PER_S * D,), jnp.float32),
            pltpu.VMEM((L,), jnp.float32),
            pltpu.SemaphoreType.DMA,
        ],
    )(_sc_body)
    pad = ((0, VP - V), (0, 0))
    code_flat, kld = f(indices, eps.reshape(B * D),
                       jnp.pad(mean_table, pad).reshape(VP * D),
                       jnp.pad(logvar_table, pad).reshape(VP * D))
    return code_flat.reshape(B, D), kld


def kernel(indices, eps, mean_table, logvar_table):
    code, kld = _run(indices.astype(jnp.int32), eps, mean_table, logvar_table)
    return (code, kld)


# speedup vs baseline: 4.2561x; 1.0963x over previous
import functools

import jax
import jax.numpy as jnp
from jax import lax
from jax.experimental import pallas as pl
from jax.experimental.pallas import tpu as pltpu
from jax.experimental.pallas import tpu_sc as plsc

V = 100
VP = 128
D = 128
B = 16384
VARIATIONAL_WEIGHT = 0.001

NC = 2
NS = 16
L = 16
NW = NC * NS
B_PER_W = B // NW
CHUNK = 128
N_CHUNKS = B_PER_W // CHUNK
R_PER_S = VP // NS


def _sc_body(idx_hbm, eps_hbm, mean_hbm, lv_hbm, code_hbm, kld_hbm,
             std_sh, g_sh,
             mean_t, std_t, g_v, idx_v, kld_v,
             eps_v0, eps_v1, code_v0, code_v1,
             a_st, b_st, g_st,
             sem_i0, sem_i1, sem_o0, sem_o1, sem_x):
    sid = lax.axis_index("s")
    wid = sid * NC + lax.axis_index("c")
    lane = lax.iota(jnp.int32, L)
    wbase = wid * B_PER_W
    eps_v = (eps_v0, eps_v1)
    code_v = (code_v0, code_v1)
    sem_i = (sem_i0, sem_i1)
    sem_o = (sem_o0, sem_o1)

    cp_idx = pltpu.async_copy(idx_hbm.at[pl.ds(wbase, B_PER_W)], idx_v, sem_x)
    cp_mean = pltpu.async_copy(mean_hbm, mean_t, sem_x)
    cp_eps0 = pltpu.async_copy(eps_hbm.at[pl.ds(wbase * D, CHUNK * D)],
                               eps_v[0], sem_i[0])

    trow = sid * R_PER_S
    pltpu.sync_copy(lv_hbm.at[pl.ds(trow * D, R_PER_S * D)], a_st)
    pltpu.sync_copy(mean_hbm.at[pl.ds(trow * D, R_PER_S * D)], b_st)

    def prow(r2, g_acc):
        acc = jnp.zeros((L,), jnp.float32)
        for j in range(D // L):
            sl = pl.ds(r2 * D + j * L, L)
            lv = a_st[sl]
            m = b_st[sl]
            s = jnp.exp(0.5 * lv)
            a_st[sl] = s
            acc = acc + (1.0 + lv - m * m - s * s)
        for _ in range(4):
            acc = acc + lax.rev(acc, (0,))
        return jnp.where(lane == r2, acc * (-0.5 * VARIATIONAL_WEIGHT), g_acc)

    g_st[...] = lax.fori_loop(0, R_PER_S, prow, jnp.zeros((L,), jnp.float32))
    pltpu.sync_copy(a_st, std_sh.at[pl.ds(trow * D, R_PER_S * D)])
    pltpu.sync_copy(g_st.at[pl.ds(0, R_PER_S)],
                    g_sh.at[pl.ds(trow, R_PER_S)])
    plsc.subcore_barrier()
    pltpu.sync_copy(std_sh, std_t)
    pltpu.sync_copy(g_sh, g_v)
    cp_idx.wait()
    cp_mean.wait()

    def compute_chunk(c, buf):
        cbase = c * CHUNK

        def group_body(g, carry):
            gbase = cbase + g * L
            tbl_vec = idx_v[pl.ds(gbase, L)]
            kld_v[pl.ds(gbase, L)] = plsc.load_gather(g_v, [tbl_vec])
            for rr in range(L):
                toff = tbl_vec[rr] * D
                roff = (g * L + rr) * D
                for j in range(D // L):
                    tsl = pl.ds(toff + j * L, L)
                    rsl = pl.ds(roff + j * L, L)
                    code_v[buf][rsl] = eps_v[buf][rsl] * std_t[tsl] + mean_t[tsl]
            return carry

        lax.fori_loop(0, CHUNK // L, group_body, 0)

    cp_in = [cp_eps0, None, None, None]
    cp_out = [None, None, None, None]
    for c in range(N_CHUNKS):
        b = c & 1
        if c + 1 < N_CHUNKS:
            nbase = (wbase + (c + 1) * CHUNK) * D
            cp_in[c + 1] = pltpu.async_copy(
                eps_hbm.at[pl.ds(nbase, CHUNK * D)], eps_v[1 - b],
                sem_i[1 - b])
        cp_in[c].wait()
        if c >= 2:
            cp_out[c - 2].wait()
        compute_chunk(c, b)
        cp_out[c] = pltpu.async_copy(
            code_v[b], code_hbm.at[pl.ds((wbase + c * CHUNK) * D, CHUNK * D)],
            sem_o[b])
    cp_out[N_CHUNKS - 2].wait()
    cp_out[N_CHUNKS - 1].wait()
    pltpu.sync_copy(kld_v, kld_hbm.at[pl.ds(wbase, B_PER_W)])


@jax.jit
def _run(indices, eps, mean_table, logvar_table):
    mesh = plsc.VectorSubcoreMesh(core_axis_name="c", subcore_axis_name="s")
    f = functools.partial(
        pl.kernel,
        out_type=(
            jax.ShapeDtypeStruct((B * D,), jnp.float32),
            jax.ShapeDtypeStruct((B,), jnp.float32),
        ),
        mesh=mesh,
        compiler_params=pltpu.CompilerParams(needs_layout_passes=False),
        scratch_types=[
            pltpu.VMEM_SHARED((VP * D,), jnp.float32),
            pltpu.VMEM_SHARED((VP,), jnp.float32),
            pltpu.VMEM((VP * D,), jnp.float32),
            pltpu.VMEM((VP * D,), jnp.float32),
            pltpu.VMEM((VP,), jnp.float32),
            pltpu.VMEM((B_PER_W,), jnp.int32),
            pltpu.VMEM((B_PER_W,), jnp.float32),
            pltpu.VMEM((CHUNK * D,), jnp.float32),
            pltpu.VMEM((CHUNK * D,), jnp.float32),
            pltpu.VMEM((CHUNK * D,), jnp.float32),
            pltpu.VMEM((CHUNK * D,), jnp.float32),
            pltpu.VMEM((R_PER_S * D,), jnp.float32),
            pltpu.VMEM((R_PER_S * D,), jnp.float32),
            pltpu.VMEM((L,), jnp.float32),
            pltpu.SemaphoreType.DMA,
            pltpu.SemaphoreType.DMA,
            pltpu.SemaphoreType.DMA,
            pltpu.SemaphoreType.DMA,
            pltpu.SemaphoreType.DMA,
        ],
    )(_sc_body)
    pad = ((0, VP - V), (0, 0))
    code_flat, kld = f(indices, eps.reshape(B * D),
                       jnp.pad(mean_table, pad).reshape(VP * D),
                       jnp.pad(logvar_table, pad).reshape(VP * D))
    return code_flat.reshape(B, D), kld


def kernel(indices, eps, mean_table, logvar_table):
    code, kld = _run(indices.astype(jnp.int32), eps, mean_table, logvar_table)
    return (code, kld)


# speedup vs baseline: 5.3486x; 1.2567x over previous
import functools

import jax
import jax.numpy as jnp
from jax import lax
from jax.experimental import pallas as pl
from jax.experimental.pallas import tpu as pltpu
from jax.experimental.pallas import tpu_sc as plsc

V = 100
VP = 128
D = 128
B = 16384
VARIATIONAL_WEIGHT = 0.001

NC = 2
NS = 16
L = 16
NW = NC * NS
B_PER_W = B // NW
CHUNK = 128
N_CHUNKS = B_PER_W // CHUNK
R_PER_S = VP // NS


def _sc_body(idx_hbm, eps_hbm, mean_hbm, lv_hbm, code_hbm, kld_hbm,
             std_sh, g_sh,
             mean_t, std_t, g_v, idx_v, kld_v,
             eps_rv, code_rv,
             a_st, b_st, g_st,
             sem_i0, sem_i1, sem_o0, sem_o1, sem_x):
    sid = lax.axis_index("s")
    wid = sid * NC + lax.axis_index("c")
    lane = lax.iota(jnp.int32, L)
    wbase = wid * B_PER_W
    CD = CHUNK * D

    cp_idx = pltpu.async_copy(idx_hbm.at[pl.ds(wbase, B_PER_W)], idx_v, sem_x)
    cp_mean = pltpu.async_copy(mean_hbm, mean_t, sem_x)
    cp_eps0 = pltpu.async_copy(eps_hbm.at[pl.ds(wbase * D, CD)],
                               eps_rv.at[pl.ds(0, CD)], sem_i0)

    trow = sid * R_PER_S
    pltpu.sync_copy(lv_hbm.at[pl.ds(trow * D, R_PER_S * D)], a_st)
    pltpu.sync_copy(mean_hbm.at[pl.ds(trow * D, R_PER_S * D)], b_st)

    def prow(r2, g_acc):
        acc = jnp.zeros((L,), jnp.float32)
        for j in range(D // L):
            sl = pl.ds(r2 * D + j * L, L)
            lv = a_st[sl]
            m = b_st[sl]
            s = jnp.exp(0.5 * lv)
            a_st[sl] = s
            acc = acc + (1.0 + lv - m * m - s * s)
        for _ in range(4):
            acc = acc + lax.rev(acc, (0,))
        return jnp.where(lane == r2, acc * (-0.5 * VARIATIONAL_WEIGHT), g_acc)

    g_st[...] = lax.fori_loop(0, R_PER_S, prow, jnp.zeros((L,), jnp.float32))
    pltpu.sync_copy(a_st, std_sh.at[pl.ds(trow * D, R_PER_S * D)])
    pltpu.sync_copy(g_st.at[pl.ds(0, R_PER_S)],
                    g_sh.at[pl.ds(trow, R_PER_S)])
    plsc.subcore_barrier()
    pltpu.sync_copy(std_sh, std_t)
    pltpu.sync_copy(g_sh, g_v)
    cp_idx.wait()
    cp_mean.wait()

    def drain_in(sem):
        pltpu.make_async_copy(eps_hbm.at[pl.ds(0, CD)],
                              eps_rv.at[pl.ds(0, CD)], sem).wait()

    def drain_out(sem):
        pltpu.make_async_copy(code_rv.at[pl.ds(0, CD)],
                              code_hbm.at[pl.ds(0, CD)], sem).wait()

    def chunk_iter(c, carry):
        par = c & 1
        boff = par * CD
        cbase = c * CHUNK

        nxt = (wbase + cbase + CHUNK) * D
        nboff = (1 - par) * CD

        @pl.when((c + 1 < N_CHUNKS) & (par == 0))
        def _():
            pltpu.async_copy(eps_hbm.at[pl.ds(nxt, CD)],
                             eps_rv.at[pl.ds(nboff, CD)], sem_i1)

        @pl.when((c + 1 < N_CHUNKS) & (par == 1))
        def _():
            pltpu.async_copy(eps_hbm.at[pl.ds(nxt, CD)],
                             eps_rv.at[pl.ds(nboff, CD)], sem_i0)

        @pl.when(par == 0)
        def _():
            drain_in(sem_i0)

        @pl.when(par == 1)
        def _():
            drain_in(sem_i1)

        @pl.when((c >= 2) & (par == 0))
        def _():
            drain_out(sem_o0)

        @pl.when((c >= 2) & (par == 1))
        def _():
            drain_out(sem_o1)

        @plsc.parallel_loop(0, CHUNK // L)
        def group_body(g):
            gbase = cbase + g * L
            tbl_vec = idx_v[pl.ds(gbase, L)]
            kld_v[pl.ds(gbase, L)] = plsc.load_gather(g_v, [tbl_vec])
            for rr in range(L):
                toff = tbl_vec[rr] * D
                roff = boff + (g * L + rr) * D
                for j in range(D // L):
                    tsl = pl.ds(toff + j * L, L)
                    rsl = pl.ds(roff + j * L, L)
                    code_rv[rsl] = eps_rv[rsl] * std_t[tsl] + mean_t[tsl]

        out = (wbase + cbase) * D

        @pl.when(par == 0)
        def _():
            pltpu.async_copy(code_rv.at[pl.ds(boff, CD)],
                             code_hbm.at[pl.ds(out, CD)], sem_o0)

        @pl.when(par == 1)
        def _():
            pltpu.async_copy(code_rv.at[pl.ds(boff, CD)],
                             code_hbm.at[pl.ds(out, CD)], sem_o1)

        return carry

    lax.fori_loop(0, N_CHUNKS, chunk_iter, 0)
    drain_out(sem_o0)
    drain_out(sem_o1)
    pltpu.sync_copy(kld_v, kld_hbm.at[pl.ds(wbase, B_PER_W)])


@jax.jit
def _run(indices, eps, mean_table, logvar_table):
    mesh = plsc.VectorSubcoreMesh(core_axis_name="c", subcore_axis_name="s")
    f = functools.partial(
        pl.kernel,
        out_type=(
            jax.ShapeDtypeStruct((B * D,), jnp.float32),
            jax.ShapeDtypeStruct((B,), jnp.float32),
        ),
        mesh=mesh,
        compiler_params=pltpu.CompilerParams(needs_layout_passes=False),
        scratch_types=[
            pltpu.VMEM_SHARED((VP * D,), jnp.float32),
            pltpu.VMEM_SHARED((VP,), jnp.float32),
            pltpu.VMEM((VP * D,), jnp.float32),
            pltpu.VMEM((VP * D,), jnp.float32),
            pltpu.VMEM((VP,), jnp.float32),
            pltpu.VMEM((B_PER_W,), jnp.int32),
            pltpu.VMEM((B_PER_W,), jnp.float32),
            pltpu.VMEM((2 * CHUNK * D,), jnp.float32),
            pltpu.VMEM((2 * CHUNK * D,), jnp.float32),
            pltpu.VMEM((R_PER_S * D,), jnp.float32),
            pltpu.VMEM((R_PER_S * D,), jnp.float32),
            pltpu.VMEM((L,), jnp.float32),
            pltpu.SemaphoreType.DMA,
            pltpu.SemaphoreType.DMA,
            pltpu.SemaphoreType.DMA,
            pltpu.SemaphoreType.DMA,
            pltpu.SemaphoreType.DMA,
        ],
    )(_sc_body)
    pad = ((0, VP - V), (0, 0))
    code_flat, kld = f(indices, eps.reshape(B * D),
                       jnp.pad(mean_table, pad).reshape(VP * D),
                       jnp.pad(logvar_table, pad).reshape(VP * D))
    return code_flat.reshape(B, D), kld


def kernel(indices, eps, mean_table, logvar_table):
    code, kld = _run(indices.astype(jnp.int32), eps, mean_table, logvar_table)
    return (code, kld)


# speedup vs baseline: 6.6862x; 1.2501x over previous
import functools

import jax
import jax.numpy as jnp
from jax import lax
from jax.experimental import pallas as pl
from jax.experimental.pallas import tpu as pltpu
from jax.experimental.pallas import tpu_sc as plsc

V = 100
VP = 128
D = 128
B = 16384
VARIATIONAL_WEIGHT = 0.001

NC = 2
NS = 16
L = 16
NW = NC * NS
B_PER_W = B // NW
CHUNK = 128
N_CHUNKS = B_PER_W // CHUNK
R_PER_S = VP // NS


def _sc_body(idx_hbm, eps_hbm, mean_hbm, lv_hbm, code_hbm, kld_hbm,
             std_sh, g_sh,
             mean_t, std_t, g_v, idx_v, kld_v,
             eps_rv, code_rv,
             a_st, b_st, g_st,
             sem_i0, sem_i1, sem_o0, sem_o1, sem_x):
    sid = lax.axis_index("s")
    wid = sid * NC + lax.axis_index("c")
    lane = lax.iota(jnp.int32, L)
    wbase = wid * B_PER_W
    CD = CHUNK * D

    cp_idx = pltpu.async_copy(idx_hbm.at[pl.ds(wbase, B_PER_W)], idx_v, sem_x)
    cp_mean = pltpu.async_copy(mean_hbm, mean_t, sem_x)
    cp_eps0 = pltpu.async_copy(eps_hbm.at[pl.ds(wbase * D, CD)],
                               eps_rv.at[pl.ds(0, CD)], sem_i0)

    trow = sid * R_PER_S
    pltpu.sync_copy(lv_hbm.at[pl.ds(trow * D, R_PER_S * D)], a_st)
    pltpu.sync_copy(mean_hbm.at[pl.ds(trow * D, R_PER_S * D)], b_st)

    def prow(r2, g_acc):
        acc = jnp.zeros((L,), jnp.float32)
        for j in range(D // L):
            sl = pl.ds(r2 * D + j * L, L)
            lv = a_st[sl]
            m = b_st[sl]
            s = jnp.exp(0.5 * lv)
            a_st[sl] = s
            acc = acc + (1.0 + lv - m * m - s * s)
        for _ in range(4):
            acc = acc + lax.rev(acc, (0,))
        return jnp.where(lane == r2, acc * (-0.5 * VARIATIONAL_WEIGHT), g_acc)

    g_st[...] = lax.fori_loop(0, R_PER_S, prow, jnp.zeros((L,), jnp.float32))
    pltpu.sync_copy(a_st, std_sh.at[pl.ds(trow * D, R_PER_S * D)])
    pltpu.sync_copy(g_st.at[pl.ds(0, R_PER_S)],
                    g_sh.at[pl.ds(trow, R_PER_S)])
    plsc.subcore_barrier()
    pltpu.sync_copy(std_sh, std_t)
    pltpu.sync_copy(g_sh, g_v)
    cp_idx.wait()
    cp_mean.wait()

    def drain_in(sem):
        pltpu.make_async_copy(eps_hbm.at[pl.ds(0, CD)],
                              eps_rv.at[pl.ds(0, CD)], sem).wait()

    def drain_out(sem):
        pltpu.make_async_copy(code_rv.at[pl.ds(0, CD)],
                              code_hbm.at[pl.ds(0, CD)], sem).wait()

    def chunk_iter(c, carry):
        par = c & 1
        boff = par * CD
        cbase = c * CHUNK

        nxt = (wbase + cbase + CHUNK) * D
        nboff = (1 - par) * CD

        @pl.when((c + 1 < N_CHUNKS) & (par == 0))
        def _():
            pltpu.async_copy(eps_hbm.at[pl.ds(nxt, CD)],
                             eps_rv.at[pl.ds(nboff, CD)], sem_i1)

        @pl.when((c + 1 < N_CHUNKS) & (par == 1))
        def _():
            pltpu.async_copy(eps_hbm.at[pl.ds(nxt, CD)],
                             eps_rv.at[pl.ds(nboff, CD)], sem_i0)

        @pl.when(par == 0)
        def _():
            drain_in(sem_i0)

        @pl.when(par == 1)
        def _():
            drain_in(sem_i1)

        @pl.when((c >= 2) & (par == 0))
        def _():
            drain_out(sem_o0)

        @pl.when((c >= 2) & (par == 1))
        def _():
            drain_out(sem_o1)

        @plsc.parallel_loop(0, CHUNK // L)
        def group_body(g):
            gbase = cbase + g * L
            tbl_vec = idx_v[pl.ds(gbase, L)]
            kld_v[pl.ds(gbase, L)] = plsc.load_gather(g_v, [tbl_vec])
            for rr in range(L):
                toff = tbl_vec[rr] * D
                roff = boff + (g * L + rr) * D
                es = [eps_rv[pl.ds(roff + j * L, L)] for j in range(D // L)]
                ss = [std_t[pl.ds(toff + j * L, L)] for j in range(D // L)]
                ms = [mean_t[pl.ds(toff + j * L, L)] for j in range(D // L)]
                for j in range(D // L):
                    code_rv[pl.ds(roff + j * L, L)] = es[j] * ss[j] + ms[j]

        out = (wbase + cbase) * D

        @pl.when(par == 0)
        def _():
            pltpu.async_copy(code_rv.at[pl.ds(boff, CD)],
                             code_hbm.at[pl.ds(out, CD)], sem_o0)

        @pl.when(par == 1)
        def _():
            pltpu.async_copy(code_rv.at[pl.ds(boff, CD)],
                             code_hbm.at[pl.ds(out, CD)], sem_o1)

        return carry

    lax.fori_loop(0, N_CHUNKS, chunk_iter, 0)
    drain_out(sem_o0)
    drain_out(sem_o1)
    pltpu.sync_copy(kld_v, kld_hbm.at[pl.ds(wbase, B_PER_W)])


@jax.jit
def _run(indices, eps, mean_table, logvar_table):
    mesh = plsc.VectorSubcoreMesh(core_axis_name="c", subcore_axis_name="s")
    f = functools.partial(
        pl.kernel,
        out_type=(
            jax.ShapeDtypeStruct((B * D,), jnp.float32),
            jax.ShapeDtypeStruct((B,), jnp.float32),
        ),
        mesh=mesh,
        compiler_params=pltpu.CompilerParams(needs_layout_passes=False),
        scratch_types=[
            pltpu.VMEM_SHARED((VP * D,), jnp.float32),
            pltpu.VMEM_SHARED((VP,), jnp.float32),
            pltpu.VMEM((VP * D,), jnp.float32),
            pltpu.VMEM((VP * D,), jnp.float32),
            pltpu.VMEM((VP,), jnp.float32),
            pltpu.VMEM((B_PER_W,), jnp.int32),
            pltpu.VMEM((B_PER_W,), jnp.float32),
            pltpu.VMEM((2 * CHUNK * D,), jnp.float32),
            pltpu.VMEM((2 * CHUNK * D,), jnp.float32),
            pltpu.VMEM((R_PER_S * D,), jnp.float32),
            pltpu.VMEM((R_PER_S * D,), jnp.float32),
            pltpu.VMEM((L,), jnp.float32),
            pltpu.SemaphoreType.DMA,
            pltpu.SemaphoreType.DMA,
            pltpu.SemaphoreType.DMA,
            pltpu.SemaphoreType.DMA,
            pltpu.SemaphoreType.DMA,
        ],
    )(_sc_body)
    pad = ((0, VP - V), (0, 0))
    code_flat, kld = f(indices, eps.reshape(B * D),
                       jnp.pad(mean_table, pad).reshape(VP * D),
                       jnp.pad(logvar_table, pad).reshape(VP * D))
    return code_flat.reshape(B, D), kld


def kernel(indices, eps, mean_table, logvar_table):
    code, kld = _run(indices.astype(jnp.int32), eps, mean_table, logvar_table)
    return (code, kld)


# speedup vs baseline: 7.4070x; 1.1078x over previous
import functools

import jax
import jax.numpy as jnp
from jax import lax
from jax.experimental import pallas as pl
from jax.experimental.pallas import tpu as pltpu
from jax.experimental.pallas import tpu_sc as plsc

V = 100
VP = 128
D = 128
B = 16384
VARIATIONAL_WEIGHT = 0.001

NC = 2
NS = 16
L = 16
NW = NC * NS
B_PER_W = B // NW
CHUNK = 128
N_CHUNKS = B_PER_W // CHUNK
R_PER_S = VP // NS


def _sc_body(idx_hbm, eps_hbm, mean_hbm, lv_hbm, code_hbm, kld_hbm,
             sm_sh, g_sh,
             sm_t, g_v, idx_v, kld_v,
             eps_rv, code_rv,
             a_st, b_st, p_st, g_st,
             sem_i0, sem_i1, sem_o0, sem_o1, sem_x):
    sid = lax.axis_index("s")
    wid = sid * NC + lax.axis_index("c")
    lane = lax.iota(jnp.int32, L)
    wbase = wid * B_PER_W
    CD = CHUNK * D

    cp_idx = pltpu.async_copy(idx_hbm.at[pl.ds(wbase, B_PER_W)], idx_v, sem_x)
    cp_eps0 = pltpu.async_copy(eps_hbm.at[pl.ds(wbase * D, CD)],
                               eps_rv.at[pl.ds(0, CD)], sem_i0)

    trow = sid * R_PER_S
    pltpu.sync_copy(lv_hbm.at[pl.ds(trow * D, R_PER_S * D)], a_st)
    pltpu.sync_copy(mean_hbm.at[pl.ds(trow * D, R_PER_S * D)], b_st)

    def prow(r2, g_acc):
        acc = jnp.zeros((L,), jnp.float32)
        for j in range(D // L):
            sl = pl.ds(r2 * D + j * L, L)
            lv = a_st[sl]
            m = b_st[sl]
            s = jnp.exp(0.5 * lv)
            p_st[pl.ds(r2 * D + j * L, L)] = plsc.bitcast(
                plsc.pack(s, m, format=plsc.PackFormat.INTERLEAVED),
                jnp.int32)
            acc = acc + (1.0 + lv - m * m - s * s)
        for _ in range(4):
            acc = acc + lax.rev(acc, (0,))
        return jnp.where(lane == r2, acc * (-0.5 * VARIATIONAL_WEIGHT), g_acc)

    g_st[...] = lax.fori_loop(0, R_PER_S, prow, jnp.zeros((L,), jnp.float32))
    pltpu.sync_copy(p_st, sm_sh.at[pl.ds(trow * D, R_PER_S * D)])
    pltpu.sync_copy(g_st.at[pl.ds(0, R_PER_S)],
                    g_sh.at[pl.ds(trow, R_PER_S)])
    plsc.subcore_barrier()
    pltpu.sync_copy(sm_sh, sm_t)
    pltpu.sync_copy(g_sh, g_v)
    cp_idx.wait()

    def drain_in(sem):
        pltpu.make_async_copy(eps_hbm.at[pl.ds(0, CD)],
                              eps_rv.at[pl.ds(0, CD)], sem).wait()

    def drain_out(sem):
        pltpu.make_async_copy(code_rv.at[pl.ds(0, CD)],
                              code_hbm.at[pl.ds(0, CD)], sem).wait()

    def chunk_iter(c, carry):
        par = c & 1
        boff = par * CD
        cbase = c * CHUNK

        nxt = (wbase + cbase + CHUNK) * D
        nboff = (1 - par) * CD

        @pl.when((c + 1 < N_CHUNKS) & (par == 0))
        def _():
            pltpu.async_copy(eps_hbm.at[pl.ds(nxt, CD)],
                             eps_rv.at[pl.ds(nboff, CD)], sem_i1)

        @pl.when((c + 1 < N_CHUNKS) & (par == 1))
        def _():
            pltpu.async_copy(eps_hbm.at[pl.ds(nxt, CD)],
                             eps_rv.at[pl.ds(nboff, CD)], sem_i0)

        @pl.when(par == 0)
        def _():
            drain_in(sem_i0)

        @pl.when(par == 1)
        def _():
            drain_in(sem_i1)

        @pl.when((c >= 2) & (par == 0))
        def _():
            drain_out(sem_o0)

        @pl.when((c >= 2) & (par == 1))
        def _():
            drain_out(sem_o1)

        @plsc.parallel_loop(0, CHUNK // L)
        def group_body(g):
            gbase = cbase + g * L
            tbl_vec = idx_v[pl.ds(gbase, L)]
            kld_v[pl.ds(gbase, L)] = plsc.load_gather(g_v, [tbl_vec])
            for rr in range(L):
                toff = tbl_vec[rr] * D
                roff = boff + (g * L + rr) * D
                es = [eps_rv[pl.ds(roff + j * L, L)] for j in range(D // L)]
                sms = [sm_t[pl.ds(toff + j * L, L)] for j in range(D // L)]
                for j in range(D // L):
                    s, m = plsc.unpack(plsc.bitcast(sms[j], jnp.bfloat16),
                                       format=plsc.PackFormat.INTERLEAVED)
                    code_rv[pl.ds(roff + j * L, L)] = es[j] * s + m

        out = (wbase + cbase) * D

        @pl.when(par == 0)
        def _():
            pltpu.async_copy(code_rv.at[pl.ds(boff, CD)],
                             code_hbm.at[pl.ds(out, CD)], sem_o0)

        @pl.when(par == 1)
        def _():
            pltpu.async_copy(code_rv.at[pl.ds(boff, CD)],
                             code_hbm.at[pl.ds(out, CD)], sem_o1)

        return carry

    lax.fori_loop(0, N_CHUNKS, chunk_iter, 0)
    drain_out(sem_o0)
    drain_out(sem_o1)
    pltpu.sync_copy(kld_v, kld_hbm.at[pl.ds(wbase, B_PER_W)])


@jax.jit
def _run(indices, eps, mean_table, logvar_table):
    mesh = plsc.VectorSubcoreMesh(core_axis_name="c", subcore_axis_name="s")
    f = functools.partial(
        pl.kernel,
        out_type=(
            jax.ShapeDtypeStruct((B * D,), jnp.float32),
            jax.ShapeDtypeStruct((B,), jnp.float32),
        ),
        mesh=mesh,
        compiler_params=pltpu.CompilerParams(needs_layout_passes=False),
        scratch_types=[
            pltpu.VMEM_SHARED((VP * D,), jnp.int32),
            pltpu.VMEM_SHARED((VP,), jnp.float32),
            pltpu.VMEM((VP * D,), jnp.int32),
            pltpu.VMEM((VP,), jnp.float32),
            pltpu.VMEM((B_PER_W,), jnp.int32),
            pltpu.VMEM((B_PER_W,), jnp.float32),
            pltpu.VMEM((2 * CHUNK * D,), jnp.float32),
            pltpu.VMEM((2 * CHUNK * D,), jnp.float32),
            pltpu.VMEM((R_PER_S * D,), jnp.float32),
            pltpu.VMEM((R_PER_S * D,), jnp.float32),
            pltpu.VMEM((R_PER_S * D,), jnp.int32),
            pltpu.VMEM((L,), jnp.float32),
            pltpu.SemaphoreType.DMA,
            pltpu.SemaphoreType.DMA,
            pltpu.SemaphoreType.DMA,
            pltpu.SemaphoreType.DMA,
            pltpu.SemaphoreType.DMA,
        ],
    )(_sc_body)
    pad = ((0, VP - V), (0, 0))
    code_flat, kld = f(indices, eps.reshape(B * D),
                       jnp.pad(mean_table, pad).reshape(VP * D),
                       jnp.pad(logvar_table, pad).reshape(VP * D))
    return code_flat.reshape(B, D), kld


def kernel(indices, eps, mean_table, logvar_table):
    code, kld = _run(indices.astype(jnp.int32), eps, mean_table, logvar_table)
    return (code, kld)
